# speedup vs baseline: 3.1565x; 3.1565x over previous
import functools

import jax
import jax.numpy as jnp
from jax import lax
from jax.experimental import pallas as pl
from jax.experimental.pallas import tpu as pltpu
from jax.experimental.pallas import tpu_sc as plsc

N = 10000
DEG = 32
F = 128
GW = 4 * F
B = N * DEG

NB = 1000
NT = N // NB

NW = 32
RPW = B // NW
CH = 400
NCHUNK = RPW // CH


def _sc_gather(table, idx_flat):
    mesh = plsc.VectorSubcoreMesh(core_axis_name="c", subcore_axis_name="s")

    @functools.partial(
        pl.kernel,
        mesh=mesh,
        out_type=jax.ShapeDtypeStruct((B, F), jnp.float32),
        scratch_types=[
            pltpu.VMEM((CH,), jnp.int32),
            pltpu.VMEM((CH, F), jnp.float32),
            pltpu.SemaphoreType.DMA,
        ],
    )
    def gather_kernel(table_hbm, idx_hbm, out_hbm, idx_v, rows_v, sem):
        wid = lax.axis_index("s") * 2 + lax.axis_index("c")
        base = wid * RPW

        def body(k, carry):
            off = pl.multiple_of(base + k * CH, 8)
            pltpu.sync_copy(idx_hbm.at[pl.ds(off, CH)], idx_v)
            pltpu.async_copy(table_hbm.at[idx_v], rows_v, sem).wait()
            pltpu.sync_copy(rows_v, out_hbm.at[pl.ds(off, CH)])
            return carry

        lax.fori_loop(0, NCHUNK, body, 0)

    return gather_kernel(table, idx_flat)


def _lstm_step(x, h, c, wih_ref, whh_ref, b_ref):
    gs = []
    for k in range(4):
        sl = slice(k * F, (k + 1) * F)
        gs.append(
            jnp.dot(x, wih_ref[:, sl], preferred_element_type=jnp.float32)
            + jnp.dot(h, whh_ref[:, sl], preferred_element_type=jnp.float32)
            + b_ref[:, sl]
        )
    i = jax.nn.sigmoid(gs[0])
    f = jax.nn.sigmoid(gs[1])
    g = jnp.tanh(gs[2])
    o = jax.nn.sigmoid(gs[3])
    c2 = f * c + i * g
    h2 = o * jnp.tanh(c2)
    return h2, c2


def _layer1_body(g_ref, feat_ref, wih_ref, whh_ref, wself_ref, wneigh_ref,
                 b_ref, bo_ref, out_ref, h_ref, c_ref):
    t = pl.program_id(1)

    @pl.when(t == 0)
    def _():
        h_ref[...] = jnp.zeros_like(h_ref)
        c_ref[...] = jnp.zeros_like(c_ref)

    h2, c2 = _lstm_step(g_ref[0], h_ref[...], c_ref[...], wih_ref, whh_ref, b_ref)
    h_ref[...] = h2
    c_ref[...] = c2

    @pl.when(t == DEG - 1)
    def _():
        out = (
            jnp.dot(feat_ref[...], wself_ref[...], preferred_element_type=jnp.float32)
            + jnp.dot(h2, wneigh_ref[...], preferred_element_type=jnp.float32)
            + bo_ref[...]
        )
        out_ref[...] = jnp.maximum(out, 0.0)


def _layer1(g, feat, wih_t, whh_t, wself_t, wneigh_t, bvec, bout):
    return pl.pallas_call(
        _layer1_body,
        grid=(NT, DEG),
        in_specs=[
            pl.BlockSpec((1, NB, F), lambda i, t: (t, i, 0)),
            pl.BlockSpec((NB, F), lambda i, t: (i, 0)),
            pl.BlockSpec((F, GW), lambda i, t: (0, 0)),
            pl.BlockSpec((F, GW), lambda i, t: (0, 0)),
            pl.BlockSpec((F, F), lambda i, t: (0, 0)),
            pl.BlockSpec((F, F), lambda i, t: (0, 0)),
            pl.BlockSpec((1, GW), lambda i, t: (0, 0)),
            pl.BlockSpec((1, F), lambda i, t: (0, 0)),
        ],
        out_specs=pl.BlockSpec((NB, F), lambda i, t: (i, 0)),
        out_shape=jax.ShapeDtypeStruct((N, F), jnp.float32),
        scratch_shapes=[
            pltpu.VMEM((NB, F), jnp.float32),
            pltpu.VMEM((NB, F), jnp.float32),
        ],
    )(g, feat, wih_t, whh_t, wself_t, wneigh_t, bvec, bout)


def _layer2_body(g_ref, feat_ref, wih_ref, whh_ref, wself_ref, wneigh_ref,
                 b_ref, bo_ref, muw_ref, mub_ref, sgw_ref, sgb_ref,
                 mu_ref, sg_ref, h_ref, c_ref, acc_ref):
    i_ = pl.program_id(0)
    t = pl.program_id(1)

    @pl.when(jnp.logical_and(i_ == 0, t == 0))
    def _():
        acc_ref[...] = jnp.zeros_like(acc_ref)

    @pl.when(t == 0)
    def _():
        h_ref[...] = jnp.zeros_like(h_ref)
        c_ref[...] = jnp.zeros_like(c_ref)

    h2, c2 = _lstm_step(g_ref[0], h_ref[...], c_ref[...], wih_ref, whh_ref, b_ref)
    h_ref[...] = h2
    c_ref[...] = c2

    @pl.when(t == DEG - 1)
    def _():
        out = (
            jnp.dot(feat_ref[...], wself_ref[...], preferred_element_type=jnp.float32)
            + jnp.dot(h2, wneigh_ref[...], preferred_element_type=jnp.float32)
            + bo_ref[...]
        )
        acc_ref[...] += jnp.sum(out, axis=0, keepdims=True)

    @pl.when(jnp.logical_and(i_ == NT - 1, t == DEG - 1))
    def _():
        x = acc_ref[...] * (1.0 / N)
        mu_ref[...] = (
            jnp.dot(x, muw_ref[...], preferred_element_type=jnp.float32) + mub_ref[...]
        )
        sg_ref[...] = (
            jnp.dot(x, sgw_ref[...], preferred_element_type=jnp.float32) + sgb_ref[...]
        )


def _layer2(g, feat, wih_t, whh_t, wself_t, wneigh_t, bvec, bout,
            muw_t, mub, sgw_t, sgb):
    rep = muw_t.shape[1]
    return pl.pallas_call(
        _layer2_body,
        grid=(NT, DEG),
        in_specs=[
            pl.BlockSpec((1, NB, F), lambda i, t: (t, i, 0)),
            pl.BlockSpec((NB, F), lambda i, t: (i, 0)),
            pl.BlockSpec((F, GW), lambda i, t: (0, 0)),
            pl.BlockSpec((F, GW), lambda i, t: (0, 0)),
            pl.BlockSpec((F, F), lambda i, t: (0, 0)),
            pl.BlockSpec((F, F), lambda i, t: (0, 0)),
            pl.BlockSpec((1, GW), lambda i, t: (0, 0)),
            pl.BlockSpec((1, F), lambda i, t: (0, 0)),
            pl.BlockSpec((F, rep), lambda i, t: (0, 0)),
            pl.BlockSpec((1, rep), lambda i, t: (0, 0)),
            pl.BlockSpec((F, rep), lambda i, t: (0, 0)),
            pl.BlockSpec((1, rep), lambda i, t: (0, 0)),
        ],
        out_specs=[
            pl.BlockSpec((1, rep), lambda i, t: (0, 0)),
            pl.BlockSpec((1, rep), lambda i, t: (0, 0)),
        ],
        out_shape=[
            jax.ShapeDtypeStruct((1, rep), jnp.float32),
            jax.ShapeDtypeStruct((1, rep), jnp.float32),
        ],
        scratch_shapes=[
            pltpu.VMEM((NB, F), jnp.float32),
            pltpu.VMEM((NB, F), jnp.float32),
            pltpu.VMEM((1, F), jnp.float32),
        ],
    )(g, feat, wih_t, whh_t, wself_t, wneigh_t, bvec, bout,
      muw_t, mub, sgw_t, sgb)


def kernel(in_feat, edge_src, lstm1_Wih, lstm1_Whh, lstm1_bih, lstm1_bhh,
           fc_self1, fc_neigh1, bias1, lstm2_Wih, lstm2_Whh, lstm2_bih,
           lstm2_bhh, fc_self2, fc_neigh2, bias2, mu_W, mu_b, sigma_W, sigma_b):
    idx = edge_src.T.reshape(-1)

    g1 = _sc_gather(in_feat, idx).reshape(DEG, N, F)
    out1 = _layer1(
        g1, in_feat,
        lstm1_Wih.T, lstm1_Whh.T, fc_self1.T, fc_neigh1.T,
        (lstm1_bih + lstm1_bhh).reshape(1, GW), bias1.reshape(1, F),
    )

    g2 = _sc_gather(out1, idx).reshape(DEG, N, F)
    mu, sigma = _layer2(
        g2, out1,
        lstm2_Wih.T, lstm2_Whh.T, fc_self2.T, fc_neigh2.T,
        (lstm2_bih + lstm2_bhh).reshape(1, GW), bias2.reshape(1, F),
        mu_W.T, mu_b.reshape(1, -1), sigma_W.T, sigma_b.reshape(1, -1),
    )
    return (mu, sigma)
